# baseline (speedup 1.0000x reference)
import functools

import jax
import jax.numpy as jnp
from jax import lax
from jax.experimental import pallas as pl
from jax.experimental.pallas import tpu as pltpu
from jax.experimental.pallas import tpu_sc as plsc

_F32 = jnp.float32


@functools.partial(
    pl.kernel,
    out_type=(
        jax.ShapeDtypeStruct((3,), _F32),
        jax.ShapeDtypeStruct((6,), _F32),
        jax.ShapeDtypeStruct((6,), _F32),
        jax.ShapeDtypeStruct((16,), _F32),
        jax.ShapeDtypeStruct((12,), _F32),
    ),
    mesh=plsc.VectorSubcoreMesh(
        core_axis_name="c", subcore_axis_name="s", num_cores=1,
        num_subcores=1),
    scratch_types=[
        pltpu.VMEM((16,), _F32),
        pltpu.VMEM((2, 16), _F32),
        pltpu.VMEM((2, 3, 16), _F32),
        pltpu.VMEM((16,), _F32),
        pltpu.VMEM((16,), _F32),
        pltpu.VMEM((16,), _F32),
        pltpu.VMEM((16,), _F32),
        pltpu.VMEM((16,), _F32),
        pltpu.SemaphoreType.DMA,
    ],
)
def _gather_kernel(x_hbm, y_hbm, z_hbm,
                   a_hbm, b_hbm, c_hbm, d_hbm, e_hbm,
                   xbuf, ybuf, zbuf, abuf, bbuf, cbuf, dbuf, ebuf, sem):
    sid = lax.axis_index("s")

    @pl.when(sid == 0)
    def _():
        pltpu.async_copy(x_hbm.at[pl.ds(0, 16)], xbuf, sem)
        pltpu.async_copy(y_hbm, ybuf, sem)
        pltpu.async_copy(z_hbm, zbuf, sem).wait()
        pltpu.make_async_copy(x_hbm.at[pl.ds(0, 16)], xbuf, sem).wait()
        pltpu.make_async_copy(y_hbm, ybuf, sem).wait()

        lane = lax.iota(jnp.int32, 16)
        vx = xbuf[...]
        vy0 = ybuf[0, :]
        vy1 = ybuf[1, :]
        vz00 = zbuf[0, 0, :]
        vz10 = zbuf[1, 0, :]

        def bcast(s):
            return jnp.full((16,), s, _F32)

        def lanechain(values):
            out = bcast(values[-1])
            for t in range(len(values) - 2, -1, -1):
                out = jnp.where(lane == t, bcast(values[t]), out)
            return out

        abuf[...] = lanechain([vx[2], vx[0], vx[1]])
        bbuf[...] = lanechain(
            [vy0[0], vy1[1], vy1[0], vy0[1], vy0[0], vy0[1]])
        cbuf[...] = lanechain(
            [vy0[1], vy0[0], vy0[2], vy1[0], vy1[2], vy1[1]])
        dbuf[...] = lanechain(
            [vz00[0], vz00[1], vz00[2], vz00[3]] * 2
            + [vz10[0], vz10[1], vz10[2], vz10[3]] * 2)
        ev = [bcast(zbuf[i, j, :][0]) for i in range(2) for j in range(3)]
        evec = ev[5]
        for t in range(4, -1, -1):
            evec = jnp.where(lane < 2 * (t + 1), ev[t], evec)
        ebuf[...] = evec

        pltpu.async_copy(abuf.at[pl.ds(0, 3)], a_hbm, sem)
        pltpu.async_copy(bbuf.at[pl.ds(0, 6)], b_hbm, sem)
        pltpu.async_copy(cbuf.at[pl.ds(0, 6)], c_hbm, sem)
        pltpu.async_copy(dbuf, d_hbm, sem)
        pltpu.async_copy(ebuf.at[pl.ds(0, 12)], e_hbm, sem).wait()
        pltpu.make_async_copy(abuf.at[pl.ds(0, 3)], a_hbm, sem).wait()
        pltpu.make_async_copy(bbuf.at[pl.ds(0, 6)], b_hbm, sem).wait()
        pltpu.make_async_copy(cbuf.at[pl.ds(0, 6)], c_hbm, sem).wait()
        pltpu.make_async_copy(dbuf, d_hbm, sem).wait()


def kernel(x, y, z):
    y2 = lax.slice(y, (0, 0), (2, 16))
    z2 = lax.slice(z, (0, 0, 0), (2, 3, 16))
    a, b, c, d, e = _gather_kernel(x, y2, z2)
    return (a, b.reshape(3, 2), c.reshape(2, 3),
            d.reshape(2, 2, 4), e.reshape(2, 3, 2))


# speedup vs baseline: 1.0693x; 1.0693x over previous
import functools

import jax
import jax.numpy as jnp
from jax import lax
from jax.experimental import pallas as pl
from jax.experimental.pallas import tpu as pltpu
from jax.experimental.pallas import tpu_sc as plsc

_F32 = jnp.float32


@functools.partial(
    pl.kernel,
    out_type=jax.ShapeDtypeStruct((64,), _F32),
    mesh=plsc.VectorSubcoreMesh(
        core_axis_name="c", subcore_axis_name="s", num_cores=1,
        num_subcores=1),
    scratch_types=[
        pltpu.VMEM((144,), _F32),
        pltpu.VMEM((64,), _F32),
    ],
)
def _gather_kernel(win_hbm, out_hbm, winbuf, obuf):
    sid = lax.axis_index("s")

    @pl.when(sid == 0)
    def _():
        pltpu.sync_copy(win_hbm, winbuf)

        lane = lax.iota(jnp.int32, 16)
        vx = winbuf[pl.ds(0, 16)]
        vy0 = winbuf[pl.ds(16, 16)]
        vy1 = winbuf[pl.ds(32, 16)]
        vz = [[winbuf[pl.ds(48 + 16 * (3 * i + j), 16)] for j in range(3)]
              for i in range(2)]

        def bcast(s):
            return jnp.full((16,), s, _F32)

        def lanechain(values):
            out = bcast(values[-1])
            for t in range(len(values) - 2, -1, -1):
                out = jnp.where(lane == t, bcast(values[t]), out)
            return out

        obuf[pl.ds(0, 16)] = lanechain([vx[2], vx[0], vx[1]])
        obuf[pl.ds(16, 16)] = lanechain(
            [vy0[0], vy1[1], vy1[0], vy0[1], vy0[0], vy0[1],
             vy0[1], vy0[0], vy0[2], vy1[0], vy1[2], vy1[1]])
        vz00, vz10 = vz[0][0], vz[1][0]
        obuf[pl.ds(32, 16)] = lanechain(
            [vz00[0], vz00[1], vz00[2], vz00[3]] * 2
            + [vz10[0], vz10[1], vz10[2], vz10[3]] * 2)
        ev = [bcast(vz[i][j][0]) for i in range(2) for j in range(3)]
        evec = ev[5]
        for t in range(4, -1, -1):
            evec = jnp.where(lane < 2 * (t + 1), ev[t], evec)
        obuf[pl.ds(48, 16)] = evec

        pltpu.sync_copy(obuf, out_hbm)


def _fmt_body(p_ref, a_ref, b_ref, c_ref, d_ref, e_ref):
    a_ref[...] = p_ref[pl.ds(0, 3)]
    for r in range(3):
        b_ref[r, :] = p_ref[pl.ds(16 + 2 * r, 2)]
    for r in range(2):
        c_ref[r, :] = p_ref[pl.ds(22 + 3 * r, 3)]
    for i in range(2):
        for j in range(2):
            d_ref[i, j, :] = p_ref[pl.ds(32 + 8 * i + 4 * j, 4)]
    for i in range(2):
        for j in range(3):
            e_ref[i, j, :] = p_ref[pl.ds(48 + 6 * i + 2 * j, 2)]


_fmt = pl.pallas_call(
    _fmt_body,
    out_shape=(
        jax.ShapeDtypeStruct((3,), _F32),
        jax.ShapeDtypeStruct((3, 2), _F32),
        jax.ShapeDtypeStruct((2, 3), _F32),
        jax.ShapeDtypeStruct((2, 2, 4), _F32),
        jax.ShapeDtypeStruct((2, 3, 2), _F32),
    ),
)


def kernel(x, y, z):
    win = jnp.concatenate(
        [lax.slice(x, (0,), (16,)), y[0, 0:16], y[1, 0:16]]
        + [z[i, j, 0:16] for i in range(2) for j in range(3)])
    packed = _gather_kernel(win)
    return _fmt(packed)


# speedup vs baseline: 1.0937x; 1.0228x over previous
import functools

import jax
import jax.numpy as jnp
from jax import lax
from jax.experimental import pallas as pl
from jax.experimental.pallas import tpu as pltpu
from jax.experimental.pallas import tpu_sc as plsc

_F32 = jnp.float32


_Y = [[16 + 0, 16 + 1, 16 + 2], [32 + 0, 32 + 1, 32 + 2]]
_Z = [[48 + 16 * (3 * i + j) for j in range(3)] for i in range(2)]

_ASSIGN = (
    [(0, 2), (1, 0), (2, 1)]
    + list(zip(range(16, 22),
               [_Y[0][0], _Y[1][1], _Y[1][0], _Y[0][1], _Y[0][0], _Y[0][1]]))
    + list(zip(range(22, 28),
               [_Y[0][1], _Y[0][0], _Y[0][2], _Y[1][0], _Y[1][2], _Y[1][1]]))
    + list(zip(range(32, 48),
               [_Z[0][0] + k for k in range(4)] * 2
               + [_Z[1][0] + k for k in range(4)] * 2))
    + list(zip(range(48, 60),
               [_Z[i][j] for i in range(2) for j in range(3)
                for _ in range(2)]))
)


@functools.partial(
    pl.kernel,
    out_type=jax.ShapeDtypeStruct((64,), _F32),
    mesh=plsc.ScalarSubcoreMesh(axis_name="c", num_cores=1),
    scratch_types=[
        pltpu.SMEM((144,), _F32),
        pltpu.SMEM((64,), _F32),
    ],
)
def _gather_kernel(win_hbm, out_hbm, winbuf, obuf):
    pltpu.sync_copy(win_hbm, winbuf)
    for dst, src in _ASSIGN:
        obuf[dst] = winbuf[src]
    pltpu.sync_copy(obuf, out_hbm)


def _fmt_body(p_ref, a_ref, b_ref, c_ref, d_ref, e_ref):
    a_ref[...] = p_ref[pl.ds(0, 3)]
    for r in range(3):
        b_ref[r, :] = p_ref[pl.ds(16 + 2 * r, 2)]
    for r in range(2):
        c_ref[r, :] = p_ref[pl.ds(22 + 3 * r, 3)]
    for i in range(2):
        for j in range(2):
            d_ref[i, j, :] = p_ref[pl.ds(32 + 8 * i + 4 * j, 4)]
    for i in range(2):
        for j in range(3):
            e_ref[i, j, :] = p_ref[pl.ds(48 + 6 * i + 2 * j, 2)]


_fmt = pl.pallas_call(
    _fmt_body,
    out_shape=(
        jax.ShapeDtypeStruct((3,), _F32),
        jax.ShapeDtypeStruct((3, 2), _F32),
        jax.ShapeDtypeStruct((2, 3), _F32),
        jax.ShapeDtypeStruct((2, 2, 4), _F32),
        jax.ShapeDtypeStruct((2, 3, 2), _F32),
    ),
)


def kernel(x, y, z):
    win = jnp.concatenate(
        [lax.slice(x, (0,), (16,)), y[0, 0:16], y[1, 0:16]]
        + [z[i, j, 0:16] for i in range(2) for j in range(3)])
    packed = _gather_kernel(win)
    return _fmt(packed)
